# Initial kernel scaffold; baseline (speedup 1.0000x reference)
import jax
import jax.numpy as jnp
from jax.experimental import pallas as pl


def kernel(words, postags, labels, sent_lengths, word_emb, pos_emb, ln_scale, ln_bias, Wx_f, Wh_f, b_f, Wx_b, Wh_b, b_b, W1, b1, W2, b2):
    raise NotImplementedError("write your pallas kernel here")


# speedup vs baseline: 2.2880x; 2.2880x over previous
import functools

import jax
import jax.numpy as jnp
from jax import lax
from jax.experimental import pallas as pl
from jax.experimental.pallas import tpu as pltpu
from jax.experimental.pallas import tpu_sc as plsc


def _sc_gather2(word_table, word_idx, pos_table, pos_idx):
    n, = word_idx.shape
    dw = word_table.shape[1]
    dp = pos_table.shape[1]
    nw = 32
    bpw = n // nw
    ch = 400
    nch = bpw // ch
    assert bpw % ch == 0 and n % nw == 0 and ch % 8 == 0

    mesh = plsc.VectorSubcoreMesh(core_axis_name="c", subcore_axis_name="s")

    @functools.partial(
        pl.kernel,
        mesh=mesh,
        out_type=[jax.ShapeDtypeStruct((n, dw), jnp.float32),
                  jax.ShapeDtypeStruct((n, dp), jnp.float32)],
        scratch_types=[
            pltpu.VMEM((ch,), jnp.int32),
            pltpu.VMEM((ch,), jnp.int32),
            pltpu.VMEM((ch, dw), jnp.float32),
            pltpu.VMEM((ch, dp), jnp.float32),
            pltpu.SemaphoreType.DMA,
            pltpu.SemaphoreType.DMA,
        ],
    )
    def k(wt_hbm, wi_hbm, pt_hbm, pi_hbm, wout_hbm, pout_hbm,
          wi_v, pi_v, wrows_v, prows_v, sem_w, sem_p):
        wid = lax.axis_index("s") * 2 + lax.axis_index("c")
        base = wid * bpw
        for c in range(nch):
            off = base + c * ch
            pltpu.sync_copy(wi_hbm.at[pl.ds(off, ch)], wi_v)
            pltpu.sync_copy(pi_hbm.at[pl.ds(off, ch)], pi_v)
            cw = pltpu.async_copy(wt_hbm.at[wi_v], wrows_v, sem_w)
            cp = pltpu.async_copy(pt_hbm.at[pi_v], prows_v, sem_p)
            cw.wait()
            cp.wait()
            pltpu.sync_copy(wrows_v, wout_hbm.at[pl.ds(off, ch)])
            pltpu.sync_copy(prows_v, pout_hbm.at[pl.ds(off, ch)])

    return k(word_table, word_idx, pos_table, pos_idx)


def _embed_norm(word_vecs, pos_vecs, ln_scale2, ln_bias2, dp):
    n, dw = word_vecs.shape
    dpad = pos_vecs.shape[1]
    din = dw + dp
    tile = 2048
    nb = n // tile

    def body(wv_ref, pv_ref, s_ref, b_ref, out_ref):
        x = jnp.concatenate([wv_ref[...], pv_ref[:, :dp]], axis=-1)
        mu = jnp.mean(x, axis=-1, keepdims=True)
        var = jnp.mean((x - mu) * (x - mu), axis=-1, keepdims=True)
        out_ref[...] = ((x - mu) / jnp.sqrt(var + 1e-5) * s_ref[...]
                        + b_ref[...])

    return pl.pallas_call(
        body,
        grid=(nb,),
        in_specs=[
            pl.BlockSpec((tile, dw), lambda i: (i, 0)),
            pl.BlockSpec((tile, dpad), lambda i: (i, 0)),
            pl.BlockSpec((1, din), lambda i: (0, 0)),
            pl.BlockSpec((1, din), lambda i: (0, 0)),
        ],
        out_specs=pl.BlockSpec((tile, din), lambda i: (i, 0)),
        out_shape=jax.ShapeDtypeStruct((n, din), jnp.float32),
    )(word_vecs, pos_vecs, ln_scale2, ln_bias2)


def _bilstm(xn, wx_s, wh_s, b_s):
    seq, bsz, din = xn.shape
    hid = wh_s.shape[1]
    g4 = 4 * hid

    def body(x_ref, wx_ref, wh_ref, b_ref, out_ref, h_ref, c_ref):
        t = pl.program_id(1)

        @pl.when(t == 0)
        def _():
            h_ref[...] = jnp.zeros_like(h_ref)
            c_ref[...] = jnp.zeros_like(c_ref)

        x = x_ref[0]
        g = (jnp.dot(x.astype(jnp.bfloat16),
                     wx_ref[0].astype(jnp.bfloat16),
                     preferred_element_type=jnp.float32)
             + jnp.dot(h_ref[...].astype(jnp.bfloat16),
                       wh_ref[0].astype(jnp.bfloat16),
                       preferred_element_type=jnp.float32)
             + b_ref[0])
        i = jax.nn.sigmoid(g[:, 0:hid])
        f = jax.nn.sigmoid(g[:, hid:2 * hid])
        gg = jnp.tanh(g[:, 2 * hid:3 * hid])
        o = jax.nn.sigmoid(g[:, 3 * hid:4 * hid])
        c = f * c_ref[...] + i * gg
        h = o * jnp.tanh(c)
        h_ref[...] = h
        c_ref[...] = c
        out_ref[0] = h

    def t_eff(d, t):
        return jnp.where(d == 0, t, seq - 1 - t)

    return pl.pallas_call(
        body,
        grid=(2, seq),
        in_specs=[
            pl.BlockSpec((1, bsz, din), lambda d, t: (t_eff(d, t), 0, 0)),
            pl.BlockSpec((1, din, g4), lambda d, t: (d, 0, 0)),
            pl.BlockSpec((1, hid, g4), lambda d, t: (d, 0, 0)),
            pl.BlockSpec((1, 1, g4), lambda d, t: (d, 0, 0)),
        ],
        out_specs=pl.BlockSpec((1, bsz, hid), lambda d, t: (t_eff(d, t), 0, d)),
        out_shape=jax.ShapeDtypeStruct((seq, bsz, 2 * hid), jnp.float32),
        scratch_shapes=[
            pltpu.VMEM((bsz, hid), jnp.float32),
            pltpu.VMEM((bsz, hid), jnp.float32),
        ],
        compiler_params=pltpu.CompilerParams(
            dimension_semantics=("arbitrary", "arbitrary")),
    )(xn, wx_s, wh_s, b_s)


def _head(ctx_flat, w1, b1_2, w2, b2_2, labels3, words3, sl2, bsz):
    n, dctx = ctx_flat.shape
    dhid = w1.shape[1]
    nlab = w2.shape[1]
    tile = 2048
    nb = n // tile

    def body(ctx_ref, w1_ref, b1_ref, w2_ref, b2_ref, lab_ref, wrd_ref,
             sl_ref, logits_ref, pred_ref, loss_ref, tc_ref, cc_ref):
        step = pl.program_id(0)

        h = (jnp.dot(ctx_ref[...].astype(jnp.bfloat16),
                     w1_ref[...].astype(jnp.bfloat16),
                     preferred_element_type=jnp.float32) + b1_ref[...])
        h = jnp.where(h > 0, h, 0.1 * h)
        logits = (jnp.dot(h.astype(jnp.bfloat16),
                          w2_ref[...].astype(jnp.bfloat16),
                          preferred_element_type=jnp.float32) + b2_ref[...])
        logits_ref[...] = logits

        m = jnp.max(logits, axis=-1, keepdims=True)
        e = jnp.exp(logits - m)
        lse = jnp.log(jnp.sum(e, axis=-1, keepdims=True)) + m
        logp = logits - lse

        lab = lab_ref[0]
        wrd = wrd_ref[0]
        mask = wrd > 0

        ids = lax.broadcasted_iota(jnp.int32, (tile, nlab), 1)
        onehot = ids == lab
        nll = -jnp.sum(jnp.where(onehot, logp, 0.0), axis=-1,
                       keepdims=True)
        part = (jnp.sum(nll * mask.astype(jnp.float32)) / bsz).reshape(1, 1)

        amax = jnp.max(logits, axis=-1, keepdims=True)
        cand = jnp.where(logits == amax, ids, nlab)
        pred = jnp.min(cand, axis=-1, keepdims=True)
        pred_ref[0] = pred

        correct = jnp.sum(((lab == pred) & mask).astype(jnp.int32)).reshape(1, 1)

        @pl.when(step == 0)
        def _():
            loss_ref[...] = jnp.zeros_like(loss_ref)
            cc_ref[...] = jnp.zeros_like(cc_ref)
            tc_ref[...] = jnp.sum(sl_ref[...]).reshape(1, 1)

        loss_ref[...] += part
        cc_ref[...] += correct

    return pl.pallas_call(
        body,
        grid=(nb,),
        in_specs=[
            pl.BlockSpec((tile, dctx), lambda i: (i, 0)),
            pl.BlockSpec((dctx, dhid), lambda i: (0, 0)),
            pl.BlockSpec((1, dhid), lambda i: (0, 0)),
            pl.BlockSpec((dhid, nlab), lambda i: (0, 0)),
            pl.BlockSpec((1, nlab), lambda i: (0, 0)),
            pl.BlockSpec((1, tile, 1), lambda i: (i, 0, 0)),
            pl.BlockSpec((1, tile, 1), lambda i: (i, 0, 0)),
            pl.BlockSpec((1, bsz), lambda i: (0, 0)),
        ],
        out_specs=[
            pl.BlockSpec((tile, nlab), lambda i: (i, 0)),
            pl.BlockSpec((1, tile, 1), lambda i: (i, 0, 0)),
            pl.BlockSpec((1, 1), lambda i: (0, 0)),
            pl.BlockSpec((1, 1), lambda i: (0, 0)),
            pl.BlockSpec((1, 1), lambda i: (0, 0)),
        ],
        out_shape=[
            jax.ShapeDtypeStruct((n, nlab), jnp.float32),
            jax.ShapeDtypeStruct((nb, tile, 1), jnp.int32),
            jax.ShapeDtypeStruct((1, 1), jnp.float32),
            jax.ShapeDtypeStruct((1, 1), jnp.int32),
            jax.ShapeDtypeStruct((1, 1), jnp.int32),
        ],
    )(ctx_flat, w1, b1_2, w2, b2_2, labels3, words3, sl2)


def kernel(words, postags, labels, sent_lengths, word_emb, pos_emb, ln_scale,
           ln_bias, Wx_f, Wh_f, b_f, Wx_b, Wh_b, b_b, W1, b1, W2, b2):
    bsz, seq = words.shape
    n = bsz * seq
    tile = 2048
    nb = n // tile

    words_tm = words.T.astype(jnp.int32)
    words_flat = words_tm.reshape(n)
    postags_flat = postags.T.reshape(n).astype(jnp.int32)
    dp = pos_emb.shape[1]
    pos_pad = jnp.pad(pos_emb, ((0, 0), (0, 128 - dp)))
    word_vecs, pos_vecs = _sc_gather2(word_emb, words_flat,
                                      pos_pad, postags_flat)

    xn = _embed_norm(word_vecs, pos_vecs,
                     ln_scale.reshape(1, -1), ln_bias.reshape(1, -1), dp)
    xn = xn.reshape(seq, bsz, -1)

    wx_s = jnp.stack([Wx_f, Wx_b])
    wh_s = jnp.stack([Wh_f, Wh_b])
    b_s = jnp.stack([b_f, b_b]).reshape(2, 1, -1)
    ctx_tm = _bilstm(xn, wx_s, wh_s, b_s)

    labels3 = labels.T.reshape(nb, tile, 1).astype(jnp.int32)
    words3 = words_flat.reshape(nb, tile, 1)
    logits_flat, pred3, loss2, tc2, cc2 = _head(
        ctx_tm.reshape(n, -1), W1, b1.reshape(1, -1), W2, b2.reshape(1, -1),
        labels3, words3, sent_lengths.reshape(1, bsz).astype(jnp.int32), bsz)

    ctx = ctx_tm.swapaxes(0, 1)
    labels_pred = pred3.reshape(seq, bsz).T
    logits_3d = logits_flat.reshape(seq, bsz, -1).swapaxes(0, 1)
    loss = loss2[0, 0]
    total_count = tc2[0, 0]
    correct_count = cc2[0, 0]
    return (labels_pred, ctx, loss, logits_3d, total_count, correct_count)


# speedup vs baseline: 2.3633x; 1.0329x over previous
import functools

import jax
import jax.numpy as jnp
from jax import lax
from jax.experimental import pallas as pl
from jax.experimental.pallas import tpu as pltpu
from jax.experimental.pallas import tpu_sc as plsc


def _sc_gather2(word_table, word_idx, pos_table, pos_idx):
    n, = word_idx.shape
    dw = word_table.shape[1]
    dp = pos_table.shape[1]
    nw = 32
    bpw = n // nw
    ch = 400
    nch = bpw // ch
    assert bpw % ch == 0 and n % nw == 0 and ch % 8 == 0

    mesh = plsc.VectorSubcoreMesh(core_axis_name="c", subcore_axis_name="s")

    @functools.partial(
        pl.kernel,
        mesh=mesh,
        out_type=[jax.ShapeDtypeStruct((n, dw), jnp.float32),
                  jax.ShapeDtypeStruct((n, dp), jnp.float32)],
        scratch_types=[
            pltpu.VMEM((bpw,), jnp.int32),
            pltpu.VMEM((bpw,), jnp.int32),
            pltpu.VMEM((ch, dw), jnp.float32),
            pltpu.VMEM((ch, dw), jnp.float32),
            pltpu.SemaphoreType.DMA,
            pltpu.SemaphoreType.DMA,
            pltpu.SemaphoreType.DMA,
            pltpu.SemaphoreType.DMA,
        ],
    )
    def k(wt_hbm, wi_hbm, pt_hbm, pi_hbm, wout_hbm, pout_hbm,
          wi_v, pi_v, buf0, buf1, g0, g1, w0, w1):
        wid = lax.axis_index("s") * 2 + lax.axis_index("c")
        base = wid * bpw
        pltpu.sync_copy(wi_hbm.at[pl.ds(base, bpw)], wi_v)
        pltpu.sync_copy(pi_hbm.at[pl.ds(base, bpw)], pi_v)

        bufs = [buf0, buf1]
        gsem = [g0, g1]
        wsem = [w0, w1]
        njobs = 2 * nch

        def start_gather(c):
            if c < nch:
                src, idx, co = wt_hbm, wi_v, c
            else:
                src, idx, co = pt_hbm, pi_v, c - nch
            return pltpu.async_copy(
                src.at[idx.at[pl.ds(co * ch, ch)]], bufs[c % 2], gsem[c % 2])

        gh = [None] * njobs
        wh = [None] * njobs
        gh[0] = start_gather(0)
        for c in range(njobs):
            if c + 1 < njobs:
                if c >= 1:
                    wh[c - 1].wait()
                gh[c + 1] = start_gather(c + 1)
            gh[c].wait()
            if c < nch:
                dst = wout_hbm.at[pl.ds(base + c * ch, ch)]
            else:
                dst = pout_hbm.at[pl.ds(base + (c - nch) * ch, ch)]
            wh[c] = pltpu.async_copy(bufs[c % 2], dst, wsem[c % 2])
        wh[njobs - 2].wait()
        wh[njobs - 1].wait()

    return k(word_table, word_idx, pos_table, pos_idx)


def _embed_norm(word_vecs, pos_vecs, ln_scale2, ln_bias2, dp):
    n, dw = word_vecs.shape
    dpad = pos_vecs.shape[1]
    din = dw + dp
    tile = 2048
    nb = n // tile

    def body(wv_ref, pv_ref, s_ref, b_ref, out_ref):
        x = jnp.concatenate([wv_ref[...], pv_ref[:, :dp]], axis=-1)
        mu = jnp.mean(x, axis=-1, keepdims=True)
        var = jnp.mean((x - mu) * (x - mu), axis=-1, keepdims=True)
        out_ref[...] = ((x - mu) / jnp.sqrt(var + 1e-5) * s_ref[...]
                        + b_ref[...])

    return pl.pallas_call(
        body,
        grid=(nb,),
        in_specs=[
            pl.BlockSpec((tile, dw), lambda i: (i, 0)),
            pl.BlockSpec((tile, dpad), lambda i: (i, 0)),
            pl.BlockSpec((1, din), lambda i: (0, 0)),
            pl.BlockSpec((1, din), lambda i: (0, 0)),
        ],
        out_specs=pl.BlockSpec((tile, din), lambda i: (i, 0)),
        out_shape=jax.ShapeDtypeStruct((n, din), jnp.float32),
    )(word_vecs, pos_vecs, ln_scale2, ln_bias2)


def _bilstm(xn, wx_s, wh_s, b_s):
    seq, bsz, din = xn.shape
    hid = wh_s.shape[1]
    g4 = 4 * hid

    def body(x_ref, wx_ref, wh_ref, b_ref, out_ref, h_ref, c_ref):
        t = pl.program_id(1)

        @pl.when(t == 0)
        def _():
            h_ref[...] = jnp.zeros_like(h_ref)
            c_ref[...] = jnp.zeros_like(c_ref)

        x = x_ref[0]
        g = (jnp.dot(x.astype(jnp.bfloat16),
                     wx_ref[0].astype(jnp.bfloat16),
                     preferred_element_type=jnp.float32)
             + jnp.dot(h_ref[...].astype(jnp.bfloat16),
                       wh_ref[0].astype(jnp.bfloat16),
                       preferred_element_type=jnp.float32)
             + b_ref[0])
        i = jax.nn.sigmoid(g[:, 0:hid])
        f = jax.nn.sigmoid(g[:, hid:2 * hid])
        gg = jnp.tanh(g[:, 2 * hid:3 * hid])
        o = jax.nn.sigmoid(g[:, 3 * hid:4 * hid])
        c = f * c_ref[...] + i * gg
        h = o * jnp.tanh(c)
        h_ref[...] = h
        c_ref[...] = c
        out_ref[0] = h

    def t_eff(d, t):
        return jnp.where(d == 0, t, seq - 1 - t)

    return pl.pallas_call(
        body,
        grid=(2, seq),
        in_specs=[
            pl.BlockSpec((1, bsz, din), lambda d, t: (t_eff(d, t), 0, 0)),
            pl.BlockSpec((1, din, g4), lambda d, t: (d, 0, 0)),
            pl.BlockSpec((1, hid, g4), lambda d, t: (d, 0, 0)),
            pl.BlockSpec((1, 1, g4), lambda d, t: (d, 0, 0)),
        ],
        out_specs=pl.BlockSpec((1, bsz, hid), lambda d, t: (t_eff(d, t), 0, d)),
        out_shape=jax.ShapeDtypeStruct((seq, bsz, 2 * hid), jnp.float32),
        scratch_shapes=[
            pltpu.VMEM((bsz, hid), jnp.float32),
            pltpu.VMEM((bsz, hid), jnp.float32),
        ],
        compiler_params=pltpu.CompilerParams(
            dimension_semantics=("arbitrary", "arbitrary")),
    )(xn, wx_s, wh_s, b_s)


def _head(ctx_flat, w1, b1_2, w2, b2_2, labels3, words3, sl2, bsz):
    n, dctx = ctx_flat.shape
    dhid = w1.shape[1]
    nlab = w2.shape[1]
    tile = 2048
    nb = n // tile

    def body(ctx_ref, w1_ref, b1_ref, w2_ref, b2_ref, lab_ref, wrd_ref,
             sl_ref, logits_ref, pred_ref, loss_ref, tc_ref, cc_ref):
        step = pl.program_id(0)

        h = (jnp.dot(ctx_ref[...].astype(jnp.bfloat16),
                     w1_ref[...].astype(jnp.bfloat16),
                     preferred_element_type=jnp.float32) + b1_ref[...])
        h = jnp.where(h > 0, h, 0.1 * h)
        logits = (jnp.dot(h.astype(jnp.bfloat16),
                          w2_ref[...].astype(jnp.bfloat16),
                          preferred_element_type=jnp.float32) + b2_ref[...])
        logits_ref[...] = logits

        m = jnp.max(logits, axis=-1, keepdims=True)
        e = jnp.exp(logits - m)
        lse = jnp.log(jnp.sum(e, axis=-1, keepdims=True)) + m
        logp = logits - lse

        lab = lab_ref[0]
        wrd = wrd_ref[0]
        mask = wrd > 0

        ids = lax.broadcasted_iota(jnp.int32, (tile, nlab), 1)
        onehot = ids == lab
        nll = -jnp.sum(jnp.where(onehot, logp, 0.0), axis=-1,
                       keepdims=True)
        part = (jnp.sum(nll * mask.astype(jnp.float32)) / bsz).reshape(1, 1)

        amax = jnp.max(logits, axis=-1, keepdims=True)
        cand = jnp.where(logits == amax, ids, nlab)
        pred = jnp.min(cand, axis=-1, keepdims=True)
        pred_ref[0] = pred

        correct = jnp.sum(((lab == pred) & mask).astype(jnp.int32)).reshape(1, 1)

        @pl.when(step == 0)
        def _():
            loss_ref[...] = jnp.zeros_like(loss_ref)
            cc_ref[...] = jnp.zeros_like(cc_ref)
            tc_ref[...] = jnp.sum(sl_ref[...]).reshape(1, 1)

        loss_ref[...] += part
        cc_ref[...] += correct

    return pl.pallas_call(
        body,
        grid=(nb,),
        in_specs=[
            pl.BlockSpec((tile, dctx), lambda i: (i, 0)),
            pl.BlockSpec((dctx, dhid), lambda i: (0, 0)),
            pl.BlockSpec((1, dhid), lambda i: (0, 0)),
            pl.BlockSpec((dhid, nlab), lambda i: (0, 0)),
            pl.BlockSpec((1, nlab), lambda i: (0, 0)),
            pl.BlockSpec((1, tile, 1), lambda i: (i, 0, 0)),
            pl.BlockSpec((1, tile, 1), lambda i: (i, 0, 0)),
            pl.BlockSpec((1, bsz), lambda i: (0, 0)),
        ],
        out_specs=[
            pl.BlockSpec((tile, nlab), lambda i: (i, 0)),
            pl.BlockSpec((1, tile, 1), lambda i: (i, 0, 0)),
            pl.BlockSpec((1, 1), lambda i: (0, 0)),
            pl.BlockSpec((1, 1), lambda i: (0, 0)),
            pl.BlockSpec((1, 1), lambda i: (0, 0)),
        ],
        out_shape=[
            jax.ShapeDtypeStruct((n, nlab), jnp.float32),
            jax.ShapeDtypeStruct((nb, tile, 1), jnp.int32),
            jax.ShapeDtypeStruct((1, 1), jnp.float32),
            jax.ShapeDtypeStruct((1, 1), jnp.int32),
            jax.ShapeDtypeStruct((1, 1), jnp.int32),
        ],
    )(ctx_flat, w1, b1_2, w2, b2_2, labels3, words3, sl2)


def kernel(words, postags, labels, sent_lengths, word_emb, pos_emb, ln_scale,
           ln_bias, Wx_f, Wh_f, b_f, Wx_b, Wh_b, b_b, W1, b1, W2, b2):
    bsz, seq = words.shape
    n = bsz * seq
    tile = 2048
    nb = n // tile

    words_tm = words.T.astype(jnp.int32)
    words_flat = words_tm.reshape(n)
    postags_flat = postags.T.reshape(n).astype(jnp.int32)
    dp = pos_emb.shape[1]
    pos_pad = jnp.pad(pos_emb, ((0, 0), (0, 128 - dp)))
    word_vecs, pos_vecs = _sc_gather2(word_emb, words_flat,
                                      pos_pad, postags_flat)

    xn = _embed_norm(word_vecs, pos_vecs,
                     ln_scale.reshape(1, -1), ln_bias.reshape(1, -1), dp)
    xn = xn.reshape(seq, bsz, -1)

    wx_s = jnp.stack([Wx_f, Wx_b])
    wh_s = jnp.stack([Wh_f, Wh_b])
    b_s = jnp.stack([b_f, b_b]).reshape(2, 1, -1)
    ctx_tm = _bilstm(xn, wx_s, wh_s, b_s)

    labels3 = labels.T.reshape(nb, tile, 1).astype(jnp.int32)
    words3 = words_flat.reshape(nb, tile, 1)
    logits_flat, pred3, loss2, tc2, cc2 = _head(
        ctx_tm.reshape(n, -1), W1, b1.reshape(1, -1), W2, b2.reshape(1, -1),
        labels3, words3, sent_lengths.reshape(1, bsz).astype(jnp.int32), bsz)

    ctx = ctx_tm.swapaxes(0, 1)
    labels_pred = pred3.reshape(seq, bsz).T
    logits_3d = logits_flat.reshape(seq, bsz, -1).swapaxes(0, 1)
    loss = loss2[0, 0]
    total_count = tc2[0, 0]
    correct_count = cc2[0, 0]
    return (labels_pred, ctx, loss, logits_3d, total_count, correct_count)


# speedup vs baseline: 2.7013x; 1.1430x over previous
import functools

import jax
import jax.numpy as jnp
from jax import lax
from jax.experimental import pallas as pl
from jax.experimental.pallas import tpu as pltpu
from jax.experimental.pallas import tpu_sc as plsc


def _sc_gather(word_table, word_idx):
    n, = word_idx.shape
    dw = word_table.shape[1]
    nw = 32
    bpw = n // nw
    ch = 200
    nch = bpw // ch
    nbuf = 4
    assert bpw % ch == 0 and n % nw == 0 and ch % 8 == 0

    mesh = plsc.VectorSubcoreMesh(core_axis_name="c", subcore_axis_name="s")

    @functools.partial(
        pl.kernel,
        mesh=mesh,
        out_type=jax.ShapeDtypeStruct((n, dw), jnp.float32),
        scratch_types=[
            pltpu.VMEM((bpw,), jnp.int32),
        ] + [pltpu.VMEM((ch, dw), jnp.float32)] * nbuf
          + [pltpu.SemaphoreType.DMA] * (2 * nbuf),
    )
    def k(wt_hbm, wi_hbm, wout_hbm, wi_v, *bufsem):
        bufs = bufsem[:nbuf]
        gsem = bufsem[nbuf:2 * nbuf]
        wsem = bufsem[2 * nbuf:]
        wid = lax.axis_index("s") * 2 + lax.axis_index("c")
        base = wid * bpw
        pltpu.sync_copy(wi_hbm.at[pl.ds(base, bpw)], wi_v)

        def start_gather(c):
            return pltpu.async_copy(
                wt_hbm.at[wi_v.at[pl.ds(c * ch, ch)]],
                bufs[c % nbuf], gsem[c % nbuf])

        gh = [None] * nch
        wh = [None] * nch
        for c in range(min(nbuf - 1, nch)):
            gh[c] = start_gather(c)
        for c in range(nch):
            nxt = c + nbuf - 1
            if nxt < nch:
                if nxt >= nbuf:
                    wh[nxt - nbuf].wait()
                gh[nxt] = start_gather(nxt)
            gh[c].wait()
            wh[c] = pltpu.async_copy(
                bufs[c % nbuf],
                wout_hbm.at[pl.ds(base + c * ch, ch)], wsem[c % nbuf])
        for c in range(max(0, nch - nbuf), nch):
            if wh[c] is not None:
                wh[c].wait()

    return k(word_table, word_idx)


def _embed_norm(word_vecs, postags3, pos_emb, ln_scale2, ln_bias2):
    n, dw = word_vecs.shape
    npos, dp = pos_emb.shape
    din = dw + dp
    tile = 2048
    nb = n // tile

    def body(wv_ref, pt_ref, pe_ref, s_ref, b_ref, out_ref):
        pt = pt_ref[0]
        ids = lax.broadcasted_iota(jnp.int32, (tile, npos), 1)
        onehot = (ids == pt).astype(jnp.float32)
        pw = pe_ref[...]
        hi_mask = jnp.uint32(0xFFFF0000)
        pe1 = lax.bitcast_convert_type(
            lax.bitcast_convert_type(pw, jnp.uint32) & hi_mask, jnp.float32)
        r1 = pw - pe1
        pe2 = lax.bitcast_convert_type(
            lax.bitcast_convert_type(r1, jnp.uint32) & hi_mask, jnp.float32)
        pe3 = r1 - pe2
        pv = (jnp.dot(onehot, pe1, preferred_element_type=jnp.float32)
              + jnp.dot(onehot, pe2, preferred_element_type=jnp.float32)
              + jnp.dot(onehot, pe3, preferred_element_type=jnp.float32))
        x = jnp.concatenate([wv_ref[...], pv], axis=-1)
        mu = jnp.mean(x, axis=-1, keepdims=True)
        var = jnp.mean((x - mu) * (x - mu), axis=-1, keepdims=True)
        out_ref[...] = ((x - mu) / jnp.sqrt(var + 1e-5) * s_ref[...]
                        + b_ref[...])

    return pl.pallas_call(
        body,
        grid=(nb,),
        in_specs=[
            pl.BlockSpec((tile, dw), lambda i: (i, 0)),
            pl.BlockSpec((1, tile, 1), lambda i: (i, 0, 0)),
            pl.BlockSpec((npos, dp), lambda i: (0, 0)),
            pl.BlockSpec((1, din), lambda i: (0, 0)),
            pl.BlockSpec((1, din), lambda i: (0, 0)),
        ],
        out_specs=pl.BlockSpec((tile, din), lambda i: (i, 0)),
        out_shape=jax.ShapeDtypeStruct((n, din), jnp.float32),
    )(word_vecs, postags3, pos_emb, ln_scale2, ln_bias2)


def _bilstm(xn, wx_s, wh_s, b_s):
    seq, bsz, din = xn.shape
    hid = wh_s.shape[1]
    g4 = 4 * hid

    def body(x_ref, wx_ref, wh_ref, b_ref, out_ref, h_ref, c_ref):
        t = pl.program_id(1)

        @pl.when(t == 0)
        def _():
            h_ref[...] = jnp.zeros_like(h_ref)
            c_ref[...] = jnp.zeros_like(c_ref)

        x = x_ref[0]
        g = (jnp.dot(x.astype(jnp.bfloat16),
                     wx_ref[0].astype(jnp.bfloat16),
                     preferred_element_type=jnp.float32)
             + jnp.dot(h_ref[...].astype(jnp.bfloat16),
                       wh_ref[0].astype(jnp.bfloat16),
                       preferred_element_type=jnp.float32)
             + b_ref[0])
        i = jax.nn.sigmoid(g[:, 0:hid])
        f = jax.nn.sigmoid(g[:, hid:2 * hid])
        gg = jnp.tanh(g[:, 2 * hid:3 * hid])
        o = jax.nn.sigmoid(g[:, 3 * hid:4 * hid])
        c = f * c_ref[...] + i * gg
        h = o * jnp.tanh(c)
        h_ref[...] = h
        c_ref[...] = c
        out_ref[0] = h

    def t_eff(d, t):
        return jnp.where(d == 0, t, seq - 1 - t)

    return pl.pallas_call(
        body,
        grid=(2, seq),
        in_specs=[
            pl.BlockSpec((1, bsz, din), lambda d, t: (t_eff(d, t), 0, 0)),
            pl.BlockSpec((1, din, g4), lambda d, t: (d, 0, 0)),
            pl.BlockSpec((1, hid, g4), lambda d, t: (d, 0, 0)),
            pl.BlockSpec((1, 1, g4), lambda d, t: (d, 0, 0)),
        ],
        out_specs=pl.BlockSpec((1, bsz, hid), lambda d, t: (t_eff(d, t), 0, d)),
        out_shape=jax.ShapeDtypeStruct((seq, bsz, 2 * hid), jnp.float32),
        scratch_shapes=[
            pltpu.VMEM((bsz, hid), jnp.float32),
            pltpu.VMEM((bsz, hid), jnp.float32),
        ],
        compiler_params=pltpu.CompilerParams(
            dimension_semantics=("arbitrary", "arbitrary")),
    )(xn, wx_s, wh_s, b_s)


def _head(ctx_flat, w1, b1_2, w2, b2_2, labels3, words3, sl2, bsz):
    n, dctx = ctx_flat.shape
    dhid = w1.shape[1]
    nlab = w2.shape[1]
    tile = 2048
    nb = n // tile

    def body(ctx_ref, w1_ref, b1_ref, w2_ref, b2_ref, lab_ref, wrd_ref,
             sl_ref, logits_ref, pred_ref, loss_ref, tc_ref, cc_ref):
        step = pl.program_id(0)

        h = (jnp.dot(ctx_ref[...].astype(jnp.bfloat16),
                     w1_ref[...].astype(jnp.bfloat16),
                     preferred_element_type=jnp.float32) + b1_ref[...])
        h = jnp.where(h > 0, h, 0.1 * h)
        logits = (jnp.dot(h.astype(jnp.bfloat16),
                          w2_ref[...].astype(jnp.bfloat16),
                          preferred_element_type=jnp.float32) + b2_ref[...])
        logits_ref[...] = logits

        m = jnp.max(logits, axis=-1, keepdims=True)
        e = jnp.exp(logits - m)
        lse = jnp.log(jnp.sum(e, axis=-1, keepdims=True)) + m
        logp = logits - lse

        lab = lab_ref[0]
        wrd = wrd_ref[0]
        mask = wrd > 0

        ids = lax.broadcasted_iota(jnp.int32, (tile, nlab), 1)
        onehot = ids == lab
        nll = -jnp.sum(jnp.where(onehot, logp, 0.0), axis=-1,
                       keepdims=True)
        part = (jnp.sum(nll * mask.astype(jnp.float32)) / bsz).reshape(1, 1)

        amax = jnp.max(logits, axis=-1, keepdims=True)
        cand = jnp.where(logits == amax, ids, nlab)
        pred = jnp.min(cand, axis=-1, keepdims=True)
        pred_ref[0] = pred

        correct = jnp.sum(((lab == pred) & mask).astype(jnp.int32)).reshape(1, 1)

        @pl.when(step == 0)
        def _():
            loss_ref[...] = jnp.zeros_like(loss_ref)
            cc_ref[...] = jnp.zeros_like(cc_ref)
            tc_ref[...] = jnp.sum(sl_ref[...]).reshape(1, 1)

        loss_ref[...] += part
        cc_ref[...] += correct

    return pl.pallas_call(
        body,
        grid=(nb,),
        in_specs=[
            pl.BlockSpec((tile, dctx), lambda i: (i, 0)),
            pl.BlockSpec((dctx, dhid), lambda i: (0, 0)),
            pl.BlockSpec((1, dhid), lambda i: (0, 0)),
            pl.BlockSpec((dhid, nlab), lambda i: (0, 0)),
            pl.BlockSpec((1, nlab), lambda i: (0, 0)),
            pl.BlockSpec((1, tile, 1), lambda i: (i, 0, 0)),
            pl.BlockSpec((1, tile, 1), lambda i: (i, 0, 0)),
            pl.BlockSpec((1, bsz), lambda i: (0, 0)),
        ],
        out_specs=[
            pl.BlockSpec((tile, nlab), lambda i: (i, 0)),
            pl.BlockSpec((1, tile, 1), lambda i: (i, 0, 0)),
            pl.BlockSpec((1, 1), lambda i: (0, 0)),
            pl.BlockSpec((1, 1), lambda i: (0, 0)),
            pl.BlockSpec((1, 1), lambda i: (0, 0)),
        ],
        out_shape=[
            jax.ShapeDtypeStruct((n, nlab), jnp.float32),
            jax.ShapeDtypeStruct((nb, tile, 1), jnp.int32),
            jax.ShapeDtypeStruct((1, 1), jnp.float32),
            jax.ShapeDtypeStruct((1, 1), jnp.int32),
            jax.ShapeDtypeStruct((1, 1), jnp.int32),
        ],
    )(ctx_flat, w1, b1_2, w2, b2_2, labels3, words3, sl2)


def kernel(words, postags, labels, sent_lengths, word_emb, pos_emb, ln_scale,
           ln_bias, Wx_f, Wh_f, b_f, Wx_b, Wh_b, b_b, W1, b1, W2, b2):
    bsz, seq = words.shape
    n = bsz * seq
    tile = 2048
    nb = n // tile

    words_tm = words.T.astype(jnp.int32)
    words_flat = words_tm.reshape(n)
    word_vecs = _sc_gather(word_emb, words_flat)

    postags3 = postags.T.reshape(nb, tile, 1).astype(jnp.int32)
    xn = _embed_norm(word_vecs, postags3, pos_emb,
                     ln_scale.reshape(1, -1), ln_bias.reshape(1, -1))
    xn = xn.reshape(seq, bsz, -1)

    wx_s = jnp.stack([Wx_f, Wx_b])
    wh_s = jnp.stack([Wh_f, Wh_b])
    b_s = jnp.stack([b_f, b_b]).reshape(2, 1, -1)
    ctx_tm = _bilstm(xn, wx_s, wh_s, b_s)

    labels3 = labels.T.reshape(nb, tile, 1).astype(jnp.int32)
    words3 = words_flat.reshape(nb, tile, 1)
    logits_flat, pred3, loss2, tc2, cc2 = _head(
        ctx_tm.reshape(n, -1), W1, b1.reshape(1, -1), W2, b2.reshape(1, -1),
        labels3, words3, sent_lengths.reshape(1, bsz).astype(jnp.int32), bsz)

    ctx = ctx_tm.swapaxes(0, 1)
    labels_pred = pred3.reshape(seq, bsz).T
    logits_3d = logits_flat.reshape(seq, bsz, -1).swapaxes(0, 1)
    loss = loss2[0, 0]
    total_count = tc2[0, 0]
    correct_count = cc2[0, 0]
    return (labels_pred, ctx, loss, logits_3d, total_count, correct_count)
